# baseline (speedup 1.0000x reference)
import functools

import jax
import jax.numpy as jnp
from jax import lax
from jax.experimental import pallas as pl
from jax.experimental.pallas import tpu as pltpu
from jax.experimental.pallas import tpu_sc as plsc

_NC = 2
_NS = 16
_NW = _NC * _NS
_L = 16


def _rsqrt16(s):
    i = lax.bitcast_convert_type(s, jnp.int32)
    i = jnp.int32(0x5F3759DF) - (i >> 1)
    y = lax.bitcast_convert_type(i, jnp.float32)
    for _ in range(3):
        y = y * (1.5 - 0.5 * s * y * y)
    return y


def _make_sc_kernel(B, D, chunk):
    b_per_w = B // _NW
    n_chunks = b_per_w // chunk
    n_idx_sub = chunk // 128
    groups_per_chunk = chunk // _L
    mesh = plsc.VectorSubcoreMesh(core_axis_name="c", subcore_axis_name="s")

    @functools.partial(
        pl.kernel,
        mesh=mesh,
        compiler_params=pltpu.CompilerParams(
            needs_layout_passes=False,
            use_tc_tiling_on_sc=False,
        ),
        out_type=jax.ShapeDtypeStruct((B, D), jnp.float32),
        scratch_types=[
            pltpu.VMEM((b_per_w // 128, 128), jnp.int32),
            pltpu.VMEM((chunk, D), jnp.float32),
            pltpu.VMEM((chunk, D), jnp.float32),
            pltpu.VMEM((chunk, D), jnp.float32),
            pltpu.SemaphoreType.DMA,
        ],
    )
    def k(trans_hbm, idx_hbm, emb_hbm, out_hbm, idx_v, trans_v, emb_v, out_v, sem):
        wid = lax.axis_index("s") * _NC + lax.axis_index("c")
        base = wid * b_per_w
        iota16 = lax.iota(jnp.int32, _L)

        for r in range(b_per_w // 128):
            pltpu.sync_copy(idx_hbm.at[pl.ds(base + r * 128, 128)], idx_v.at[r])

        for c in range(n_chunks):
            row0 = base + c * chunk
            copies = []
            for r in range(n_idx_sub):
                copies.append(
                    pltpu.async_copy(
                        trans_hbm.at[idx_v.at[c * n_idx_sub + r]],
                        trans_v.at[pl.ds(r * 128, 128)],
                        sem,
                    )
                )
            copies.append(pltpu.async_copy(emb_hbm.at[pl.ds(row0, chunk)], emb_v, sem))
            for cp in copies:
                cp.wait()

            def group(g, carry):
                rows = g * _L + iota16
                accs = [jnp.zeros((_L,), jnp.float32)] * 8
                for j in range(D):
                    col = jnp.full((_L,), j, jnp.int32)
                    v = plsc.load_gather(emb_v, [rows, col]) + plsc.load_gather(
                        trans_v, [rows, col]
                    )
                    accs[j % 8] = accs[j % 8] + v * v
                tot = (accs[0] + accs[1]) + (accs[2] + accs[3])
                tot = tot + (accs[4] + accs[5]) + (accs[6] + accs[7])
                y = _rsqrt16(tot)
                inv = 1.0 / jnp.maximum(tot * y, 1e-12)
                for j in range(D):
                    col = jnp.full((_L,), j, jnp.int32)
                    v = plsc.load_gather(emb_v, [rows, col]) + plsc.load_gather(
                        trans_v, [rows, col]
                    )
                    plsc.store_scatter(out_v, [rows, col], v * inv)
                return carry

            lax.fori_loop(0, groups_per_chunk, group, 0)
            pltpu.sync_copy(out_v, out_hbm.at[pl.ds(row0, chunk)])

    return k


def kernel(embeddings, operator_idxs, entity_list, relation_dim, entity_dim, flag, rel_id, translations):
    B, D = embeddings.shape
    k = _make_sc_kernel(B, D, chunk=256)
    return k(translations, operator_idxs, embeddings)


# speedup vs baseline: 2.7779x; 2.7779x over previous
import functools

import jax
import jax.numpy as jnp
from jax import lax
from jax.experimental import pallas as pl
from jax.experimental.pallas import tpu as pltpu
from jax.experimental.pallas import tpu_sc as plsc

_NC = 2
_NS = 16
_NW = _NC * _NS
_L = 16


def _rsqrt16(s):
    i = lax.bitcast_convert_type(s, jnp.int32)
    i = jnp.int32(0x5F3759DF) - (i >> 1)
    y = lax.bitcast_convert_type(i, jnp.float32)
    for _ in range(3):
        y = y * (1.5 - 0.5 * s * y * y)
    return y


def _make_sc_kernel(B, D, chunk):
    b_per_w = B // _NW
    n_chunks = b_per_w // chunk
    n_idx_sub = chunk // 128
    groups_per_chunk = chunk // _L
    mesh = plsc.VectorSubcoreMesh(core_axis_name="c", subcore_axis_name="s")

    @functools.partial(
        pl.kernel,
        mesh=mesh,
        compiler_params=pltpu.CompilerParams(
            needs_layout_passes=False,
            use_tc_tiling_on_sc=False,
        ),
        out_type=jax.ShapeDtypeStruct((B, D), jnp.float32),
        scratch_types=[
            pltpu.VMEM((b_per_w // 128, 128), jnp.int32),
            pltpu.VMEM((chunk, D), jnp.float32),
            pltpu.VMEM((chunk, D), jnp.float32),
            pltpu.VMEM((chunk, D), jnp.float32),
            pltpu.SemaphoreType.DMA,
        ],
    )
    def k(trans_hbm, idx_hbm, emb_hbm, out_hbm, idx_v, trans_v, emb_v, out_v, sem):
        wid = lax.axis_index("s") * _NC + lax.axis_index("c")
        base = wid * b_per_w
        iota16 = lax.iota(jnp.int32, _L)

        for r in range(b_per_w // 128):
            pltpu.sync_copy(idx_hbm.at[pl.ds(base + r * 128, 128)], idx_v.at[r])

        for c in range(n_chunks):
            row0 = base + c * chunk
            copies = []
            for r in range(n_idx_sub):
                copies.append(
                    pltpu.async_copy(
                        trans_hbm.at[idx_v.at[c * n_idx_sub + r]],
                        trans_v.at[pl.ds(r * 128, 128)],
                        sem,
                    )
                )
            copies.append(pltpu.async_copy(emb_hbm.at[pl.ds(row0, chunk)], emb_v, sem))
            for cp in copies:
                cp.wait()

            def group(g, carry):
                rows = g * _L + iota16
                accs = [jnp.zeros((_L,), jnp.float32)] * 8
                for j in range(D):
                    col = (iota16 + j) & (D - 1)
                    v = plsc.load_gather(emb_v, [rows, col]) + plsc.load_gather(
                        trans_v, [rows, col]
                    )
                    accs[j % 8] = accs[j % 8] + v * v
                tot = (accs[0] + accs[1]) + (accs[2] + accs[3])
                tot = tot + (accs[4] + accs[5]) + (accs[6] + accs[7])
                y = _rsqrt16(tot)
                inv = 1.0 / jnp.maximum(tot * y, 1e-12)
                for j in range(D):
                    col = (iota16 + j) & (D - 1)
                    v = plsc.load_gather(emb_v, [rows, col]) + plsc.load_gather(
                        trans_v, [rows, col]
                    )
                    plsc.store_scatter(out_v, [rows, col], v * inv)
                return carry

            lax.fori_loop(0, groups_per_chunk, group, 0)
            pltpu.sync_copy(out_v, out_hbm.at[pl.ds(row0, chunk)])

    return k


def kernel(embeddings, operator_idxs, entity_list, relation_dim, entity_dim, flag, rel_id, translations):
    B, D = embeddings.shape
    k = _make_sc_kernel(B, D, chunk=256)
    return k(translations, operator_idxs, embeddings)
